# baseline (speedup 1.0000x reference)
import functools

import jax
import jax.numpy as jnp
from jax import lax
from jax.experimental import pallas as pl
from jax.experimental.pallas import tpu as pltpu
from jax.experimental.pallas import tpu_sc as plsc

N, D = 100000, 128
BS = 10000
G = 16
NW = 32
SLAB = 3128


def _scales_body(w_ref, s_ref):
    for g in range(4):
        eps = 0.5 * (g + 1)
        gw = w_ref[g * 1000:g * 1000 + G, :]
        norm = jnp.sqrt(jnp.sum(gw * gw, axis=1, keepdims=True))
        s_ref[g * G:(g + 1) * G, :] = eps / jnp.maximum(norm, eps)


def _scales(W):
    return pl.pallas_call(
        _scales_body,
        grid=(1,),
        in_specs=[pl.BlockSpec((3072, D), lambda i: (0, 0))],
        out_specs=pl.BlockSpec((64, 1), lambda i: (0, 0)),
        out_shape=jax.ShapeDtypeStruct((64, 1), jnp.float32),
    )(W)


def _wout_sc_body(w_hbm, scales_hbm, wout_hbm, rows_v, sv):
    cid = lax.axis_index("c")
    sid = lax.axis_index("s")
    wid = sid * 2 + cid
    base = wid * SLAB
    tail = N - (NW - 1) * SLAB

    @pl.when(wid < NW - 1)
    def _():
        pltpu.sync_copy(w_hbm.at[pl.ds(base, SLAB)],
                        wout_hbm.at[pl.ds(base, SLAB)])

    @pl.when(wid == NW - 1)
    def _():
        pltpu.sync_copy(w_hbm.at[pl.ds((NW - 1) * SLAB, tail)],
                        wout_hbm.at[pl.ds((NW - 1) * SLAB, tail)])

    @pl.when(wid == 0)
    def _():
        for g in range(4):
            pltpu.sync_copy(scales_hbm.at[pl.ds(g * G, G)], sv)
            pltpu.sync_copy(w_hbm.at[pl.ds(g * 1000, G)], rows_v)
            for r in range(G):
                s = plsc.load_gather(sv, [jnp.full((16,), r, jnp.int32)])
                for k in range(D // 16):
                    rows_v[r, pl.ds(k * 16, 16)] = (
                        rows_v[r, pl.ds(k * 16, 16)] * s)
            pltpu.sync_copy(rows_v, wout_hbm.at[pl.ds(g * 1000, G)])


_wout_sc = functools.partial(
    pl.kernel,
    out_type=jax.ShapeDtypeStruct((N, D), jnp.float32),
    mesh=plsc.VectorSubcoreMesh(core_axis_name="c", subcore_axis_name="s"),
    scratch_types=[
        pltpu.VMEM((G, D), jnp.float32),
        pltpu.VMEM((G,), jnp.float32),
    ],
    compiler_params=pltpu.CompilerParams(needs_layout_passes=False),
)(_wout_sc_body)


def _out_body(x_ref, w_ref, o_ref):
    pid = pl.program_id(0)
    w = w_ref[...]
    o_ref[...] = x_ref[...] + w

    @pl.when(pid == 0)
    def _():
        for g in range(4):
            off = g * 1000
            eps = 0.5 * (g + 1)
            gw = w[off:off + G, :]
            norm = jnp.sqrt(jnp.sum(gw * gw, axis=1, keepdims=True))
            gn = gw * (eps / jnp.maximum(norm, eps))
            o_ref[off:off + G, :] = x_ref[off:off + G, :] + gn


def _out_tc(x, W):
    return pl.pallas_call(
        _out_body,
        grid=(N // BS,),
        in_specs=[
            pl.BlockSpec((BS, D), lambda i: (i, 0)),
            pl.BlockSpec((BS, D), lambda i: (i, 0)),
        ],
        out_specs=pl.BlockSpec((BS, D), lambda i: (i, 0)),
        out_shape=jax.ShapeDtypeStruct((N, D), jnp.float32),
        compiler_params=pltpu.CompilerParams(
            dimension_semantics=("parallel",),
        ),
    )(x, W)


def kernel(x, W):
    scales = _scales(W).reshape(64)
    Wout = _wout_sc(W, scales)
    out = _out_tc(x, W)
    return (out, Wout)


# speedup vs baseline: 15.5722x; 15.5722x over previous
import functools

import jax
import jax.numpy as jnp
from jax import lax
from jax.experimental import pallas as pl
from jax.experimental.pallas import tpu as pltpu
from jax.experimental.pallas import tpu_sc as plsc

N, D = 100000, 128
BS = 10000
G = 16
NW = 32
SLAB = 3128


def _scales_body(w_ref, s_ref):
    for g in range(4):
        eps = 0.5 * (g + 1)
        gw = w_ref[g * 1000:g * 1000 + G, :]
        norm = jnp.sqrt(jnp.sum(gw * gw, axis=1, keepdims=True))
        s_ref[g * G:(g + 1) * G, :] = eps / jnp.maximum(norm, eps)


def _scales(W):
    return pl.pallas_call(
        _scales_body,
        grid=(1,),
        in_specs=[pl.BlockSpec((3072, D), lambda i: (0, 0))],
        out_specs=pl.BlockSpec((64, 1), lambda i: (0, 0)),
        out_shape=jax.ShapeDtypeStruct((64, 1), jnp.float32),
    )(W)


CH = 400
NJ = SLAB // CH
LAST = SLAB - (NJ) * CH
TAIL_LAST = N - (NW - 1) * SLAB - NJ * CH


def _wout_sc_body(w_hbm, scales_hbm, wout_hbm, buf0, buf1, rows_v, sv,
                  rs0, rs1, ws0, ws1):
    cid = lax.axis_index("c")
    sid = lax.axis_index("s")
    wid = sid * 2 + cid
    base = wid * SLAB
    bufs, rsems, wsems = [buf0, buf1], [rs0, rs1], [ws0, ws1]
    writes = {}
    for j in range(NJ):
        b = j % 2
        if j >= 2:
            writes[j - 2].wait()
        rd = pltpu.make_async_copy(
            w_hbm.at[pl.ds(base + j * CH, CH)], bufs[b], rsems[b])
        rd.start()
        rd.wait()
        wr = pltpu.make_async_copy(
            bufs[b], wout_hbm.at[pl.ds(base + j * CH, CH)], wsems[b])
        wr.start()
        writes[j] = wr
    writes[NJ - 2].wait()
    writes[NJ - 1].wait()
    for tlen, pred in ((LAST, wid < NW - 1), (TAIL_LAST, wid == NW - 1)):
        @pl.when(pred)
        def _(tlen=tlen):
            pltpu.sync_copy(w_hbm.at[pl.ds(base + NJ * CH, tlen)],
                            buf0.at[:tlen])
            pltpu.sync_copy(buf0.at[:tlen],
                            wout_hbm.at[pl.ds(base + NJ * CH, tlen)])

    @pl.when(wid == 0)
    def _():
        for g in range(4):
            pltpu.sync_copy(scales_hbm.at[pl.ds(g * G, G)], sv)
            pltpu.sync_copy(w_hbm.at[pl.ds(g * 1000, G)], rows_v)
            for r in range(G):
                s = plsc.load_gather(sv, [jnp.full((16,), r, jnp.int32)])
                for k in range(D // 16):
                    rows_v[r, pl.ds(k * 16, 16)] = (
                        rows_v[r, pl.ds(k * 16, 16)] * s)
            pltpu.sync_copy(rows_v, wout_hbm.at[pl.ds(g * 1000, G)])


_wout_sc = functools.partial(
    pl.kernel,
    out_type=jax.ShapeDtypeStruct((N, D), jnp.float32),
    mesh=plsc.VectorSubcoreMesh(core_axis_name="c", subcore_axis_name="s"),
    scratch_types=[
        pltpu.VMEM((CH, D), jnp.float32),
        pltpu.VMEM((CH, D), jnp.float32),
        pltpu.VMEM((G, D), jnp.float32),
        pltpu.VMEM((G,), jnp.float32),
        pltpu.SemaphoreType.DMA,
        pltpu.SemaphoreType.DMA,
        pltpu.SemaphoreType.DMA,
        pltpu.SemaphoreType.DMA,
    ],
    compiler_params=pltpu.CompilerParams(needs_layout_passes=False),
)(_wout_sc_body)


def _out_body(x_ref, w_ref, o_ref):
    pid = pl.program_id(0)
    w = w_ref[...]
    o_ref[...] = x_ref[...] + w

    @pl.when(pid == 0)
    def _():
        for g in range(4):
            off = g * 1000
            eps = 0.5 * (g + 1)
            gw = w[off:off + G, :]
            norm = jnp.sqrt(jnp.sum(gw * gw, axis=1, keepdims=True))
            gn = gw * (eps / jnp.maximum(norm, eps))
            o_ref[off:off + G, :] = x_ref[off:off + G, :] + gn


def _out_tc(x, W):
    return pl.pallas_call(
        _out_body,
        grid=(N // BS,),
        in_specs=[
            pl.BlockSpec((BS, D), lambda i: (i, 0)),
            pl.BlockSpec((BS, D), lambda i: (i, 0)),
        ],
        out_specs=pl.BlockSpec((BS, D), lambda i: (i, 0)),
        out_shape=jax.ShapeDtypeStruct((N, D), jnp.float32),
        compiler_params=pltpu.CompilerParams(
            dimension_semantics=("parallel",),
        ),
    )(x, W)


def kernel(x, W):
    scales = _scales(W).reshape(64)
    Wout = _wout_sc(W, scales)
    out = _out_tc(x, W)
    return (out, Wout)


# speedup vs baseline: 17.8202x; 1.1444x over previous
import functools

import jax
import jax.numpy as jnp
from jax import lax
from jax.experimental import pallas as pl
from jax.experimental.pallas import tpu as pltpu
from jax.experimental.pallas import tpu_sc as plsc

N, D = 100000, 128
BS = 10000
G = 16


def _nw_sc_body(w_hbm, nw_hbm, rows_v):
    cid = lax.axis_index("c")
    sid = lax.axis_index("s")
    wid = sid * 2 + cid

    @pl.when(wid < 4)
    def _():
        src = pl.multiple_of(wid * 1000, 8)
        dst = pl.multiple_of(wid * G, 8)
        pltpu.sync_copy(w_hbm.at[pl.ds(src, G)], rows_v)
        eps = (wid.astype(jnp.float32) + 1.0) * 0.5
        rows = lax.iota(jnp.int32, G)
        ss = jnp.zeros((G,), jnp.float32)
        for c in range(D):
            col = jnp.full((G,), c, jnp.int32)
            v = plsc.load_gather(rows_v, [rows, col])
            ss = ss + v * v
        i = plsc.bitcast(ss, jnp.int32)
        i = jnp.int32(0x5F3759DF) - (i >> 1)
        r = plsc.bitcast(i, jnp.float32)
        for _ in range(3):
            r = r * (1.5 - 0.5 * ss * r * r)
        scale = jnp.where(ss <= eps * eps, jnp.float32(1.0), eps * r)
        for c in range(D):
            col = jnp.full((G,), c, jnp.int32)
            v = plsc.load_gather(rows_v, [rows, col])
            plsc.store_scatter(rows_v, [rows, col], v * scale)
        pltpu.sync_copy(rows_v, nw_hbm.at[pl.ds(dst, G)])


_nw_sc = functools.partial(
    pl.kernel,
    out_type=jax.ShapeDtypeStruct((4 * G, D), jnp.float32),
    mesh=plsc.VectorSubcoreMesh(core_axis_name="c", subcore_axis_name="s"),
    scratch_types=[pltpu.VMEM((G, D), jnp.float32)],
    compiler_params=pltpu.CompilerParams(needs_layout_passes=False),
)(_nw_sc_body)


def _fused_body(x_ref, w_ref, nw_ref, o_ref, wout_ref):
    pid = pl.program_id(0)
    w = w_ref[...]
    wout_ref[...] = w
    o_ref[...] = x_ref[...] + w

    @pl.when(pid == 0)
    def _():
        for g in range(4):
            off = g * 1000
            nwg = nw_ref[g * G:(g + 1) * G, :]
            wout_ref[off:off + G, :] = nwg
            o_ref[off:off + G, :] = x_ref[off:off + G, :] + nwg


def _fused_tc(x, W, nw):
    return pl.pallas_call(
        _fused_body,
        grid=(N // BS,),
        in_specs=[
            pl.BlockSpec((BS, D), lambda i: (i, 0)),
            pl.BlockSpec((BS, D), lambda i: (i, 0)),
            pl.BlockSpec((4 * G, D), lambda i: (0, 0)),
        ],
        out_specs=[
            pl.BlockSpec((BS, D), lambda i: (i, 0)),
            pl.BlockSpec((BS, D), lambda i: (i, 0)),
        ],
        out_shape=[
            jax.ShapeDtypeStruct((N, D), jnp.float32),
            jax.ShapeDtypeStruct((N, D), jnp.float32),
        ],
        compiler_params=pltpu.CompilerParams(
            dimension_semantics=("parallel",),
        ),
    )(x, W, nw)


def kernel(x, W):
    nw = _nw_sc(W)
    out, Wout = _fused_tc(x, W, nw)
    return (out, Wout)


# speedup vs baseline: 24.6705x; 1.3844x over previous
import jax
import jax.numpy as jnp
from jax.experimental import pallas as pl
from jax.experimental.pallas import tpu as pltpu

N, D = 100000, 128
BS = 10000
G = 16


def _body(x_ref, w_ref, out_ref, wout_ref):
    pid = pl.program_id(0)
    w = w_ref[...]
    wout_ref[...] = w
    out_ref[...] = x_ref[...] + w

    @pl.when(pid == 0)
    def _():
        for g in range(4):
            off = g * 1000
            eps = 0.5 * (g + 1)
            gw = w[off:off + G, :]
            norm = jnp.sqrt(jnp.sum(gw * gw, axis=1, keepdims=True))
            gn = gw * (eps / jnp.maximum(norm, eps))
            wout_ref[off:off + G, :] = gn
            out_ref[off:off + G, :] = x_ref[off:off + G, :] + gn


def kernel(x, W):
    out, Wout = pl.pallas_call(
        _body,
        grid=(N // BS,),
        in_specs=[
            pl.BlockSpec((BS, D), lambda i: (i, 0)),
            pl.BlockSpec((BS, D), lambda i: (i, 0)),
        ],
        out_specs=[
            pl.BlockSpec((BS, D), lambda i: (i, 0)),
            pl.BlockSpec((BS, D), lambda i: (i, 0)),
        ],
        out_shape=[
            jax.ShapeDtypeStruct((N, D), jnp.float32),
            jax.ShapeDtypeStruct((N, D), jnp.float32),
        ],
        compiler_params=pltpu.CompilerParams(
            dimension_semantics=("parallel",),
        ),
    )(x, W)
    return (out, Wout)
